# baseline (speedup 1.0000x reference)
import math

import jax
import jax.numpy as jnp
from jax.experimental import pallas as pl
from jax.experimental.pallas import tpu as pltpu

_BLK = 512


def _prep_body(tab_ref, logit_ref, tab2_ref, tnorm_ref, gl_ref):
    d = tab_ref.shape[0]
    c = tab_ref.shape[1]
    tab = tab_ref[...]
    tab2_ref[...] = tab + tab
    tnorm_ref[...] = jnp.sum(tab * tab, axis=1)[None, :]
    mx = jnp.max(tab, axis=1, keepdims=True)
    ex = jnp.exp(tab - mx)
    ls = tab - (jnp.log(jnp.sum(ex, axis=1, keepdims=True)) + mx)
    lg = logit_ref[...]
    ml = jnp.max(lg)
    llc = lg - (jnp.log(jnp.sum(jnp.exp(lg - ml))) + ml)
    llc_col = (llc * (-1.0 / math.log(2.0))).reshape(d, 1)
    gl_ref[:, :c] = tab.astype(jnp.bfloat16)
    gl_ref[:, c:2 * c] = ls.astype(jnp.bfloat16)
    gl_ref[:, 2 * c:2 * c + 1] = llc_col.astype(jnp.bfloat16)
    gl_ref[:, 2 * c + 1:] = jnp.zeros((d, 7), jnp.bfloat16)


def _vq_body(p_ref, tab2_ref, tnorm_ref, gl_ref, lpmf_ref, pq_ref, bit_ref):
    i = pl.program_id(0)
    c = tab2_ref.shape[1]

    p = jnp.clip(p_ref[...], -1.0, 1.0)
    scores = jax.lax.dot_general(
        p, tab2_ref[...], (((1,), (1,)), ((), ())),
        preferred_element_type=jnp.float32)
    neg = scores - tnorm_ref[...]
    idx = jnp.argmax(neg, axis=1)
    iota = jax.lax.broadcasted_iota(jnp.int32, neg.shape, 1)
    oh = (iota == idx[:, None]).astype(jnp.float32).astype(jnp.bfloat16)

    qg = jax.lax.dot_general(
        oh, gl_ref[...], (((1,), (0,)), ((), ())),
        preferred_element_type=jnp.float32)
    pq_ref[...] = qg[:, :c]
    lpmf_ref[...] = qg[:, c:2 * c]
    bit_blk = jnp.sum(qg[:, 2 * c:])

    @pl.when(i == 0)
    def _():
        bit_ref[0, 0] = bit_blk

    @pl.when(i > 0)
    def _():
        bit_ref[0, 0] += bit_blk


def kernel(params, param_table, logits):
    a, b, c = params.shape
    d = param_table.shape[0]
    tokens = a * b
    p2 = params.reshape(tokens, c)
    lg2 = logits.reshape(1, d)
    grid = tokens // _BLK

    tab2, tnorm, gl = pl.pallas_call(
        _prep_body,
        out_shape=[
            jax.ShapeDtypeStruct((d, c), jnp.float32),
            jax.ShapeDtypeStruct((1, d), jnp.float32),
            jax.ShapeDtypeStruct((d, 2 * c + 8), jnp.bfloat16),
        ],
    )(param_table, lg2)

    lpmf, pq, bit = pl.pallas_call(
        _vq_body,
        grid=(grid,),
        in_specs=[
            pl.BlockSpec((_BLK, c), lambda i: (i, 0)),
            pl.BlockSpec((d, c), lambda i: (0, 0)),
            pl.BlockSpec((1, d), lambda i: (0, 0)),
            pl.BlockSpec((d, 2 * c + 8), lambda i: (0, 0)),
        ],
        out_specs=[
            pl.BlockSpec((_BLK, c), lambda i: (i, 0)),
            pl.BlockSpec((_BLK, c), lambda i: (i, 0)),
            pl.BlockSpec(memory_space=pltpu.SMEM),
        ],
        out_shape=[
            jax.ShapeDtypeStruct((tokens, c), jnp.float32),
            jax.ShapeDtypeStruct((tokens, c), jnp.float32),
            jax.ShapeDtypeStruct((1, 1), jnp.float32),
        ],
        compiler_params=pltpu.CompilerParams(
            dimension_semantics=("arbitrary",),
        ),
    )(p2, tab2, tnorm, gl)

    return (lpmf.reshape(a, b, c), pq.reshape(a, b, c), bit[0, 0])


# speedup vs baseline: 1.0172x; 1.0172x over previous
import math

import jax
import jax.numpy as jnp
from jax.experimental import pallas as pl
from jax.experimental.pallas import tpu as pltpu

_BLK = 512
_H = 256


def _prep_body(tab_ref, logit_ref, gl_ref, llcs_ref):
    c = tab_ref.shape[1]
    tab = tab_ref[...]
    mx = jnp.max(tab, axis=1, keepdims=True)
    ex = jnp.exp(tab - mx)
    ls = tab - (jnp.log(jnp.sum(ex, axis=1, keepdims=True)) + mx)
    lg = logit_ref[...]
    ml = jnp.max(lg)
    llc = lg - (jnp.log(jnp.sum(jnp.exp(lg - ml))) + ml)
    llcs_ref[...] = llc * (-1.0 / math.log(2.0))
    gl_ref[:, :c] = tab.astype(jnp.bfloat16)
    gl_ref[:, c:] = ls.astype(jnp.bfloat16)


def _vq_body(p_ref, pn_ref, tn_ref, tab_hbm, gl_hbm, llcs_hbm,
             lpmf_ref, pq_ref, bit_ref,
             tab_v, gl_v, llcs_v, cnt_v, sem):
    i = pl.program_id(0)
    nblk = pl.num_programs(0)
    d = tab_v.shape[0]
    c = tab_v.shape[1]

    @pl.when(i == 0)
    def _load_tables():
        c1 = pltpu.make_async_copy(tab_hbm, tab_v, sem)
        c2 = pltpu.make_async_copy(gl_hbm, gl_v, sem)
        c3 = pltpu.make_async_copy(llcs_hbm, llcs_v, sem)
        c1.start(); c2.start(); c3.start()
        c1.wait(); c2.wait(); c3.wait()
        cnt_v[...] = jnp.zeros((1, d), jnp.float32)

    tn = tn_ref[...]
    for h in range(_BLK // _H):
        p = jnp.clip(p_ref[pl.ds(h * _H, _H), :], -1.0, 1.0)
        pn = pn_ref[pl.ds(h * _H, _H), :]
        scores = jax.lax.dot_general(
            p, tab_v[...], (((1,), (1,)), ((), ())),
            preferred_element_type=jnp.float32)
        dist = (tn + pn) - 2.0 * scores
        idx = jnp.argmin(dist, axis=1)
        iota = jax.lax.broadcasted_iota(jnp.int32, dist.shape, 1)
        ohf = (iota == idx[:, None]).astype(jnp.float32)
        cnt_v[...] += jnp.sum(ohf, axis=0, keepdims=True)
        qg = jax.lax.dot_general(
            ohf.astype(jnp.bfloat16), gl_v[...], (((1,), (0,)), ((), ())),
            preferred_element_type=jnp.float32)
        pq_ref[pl.ds(h * _H, _H), :] = qg[:, :c]
        lpmf_ref[pl.ds(h * _H, _H), :] = qg[:, c:]

    @pl.when(i == nblk - 1)
    def _finish():
        bit_ref[0, 0] = jnp.sum(cnt_v[...] * llcs_v[...])


def kernel(params, param_table, logits):
    a, b, c = params.shape
    d = param_table.shape[0]
    tokens = a * b
    p2 = params.reshape(tokens, c)
    lg2 = logits.reshape(1, d)
    grid = tokens // _BLK

    pclip = jnp.clip(p2, -1.0, 1.0)
    pn = jnp.sum(pclip ** 2, axis=-1).reshape(tokens, 1)
    tn = jnp.sum(param_table ** 2, axis=-1).reshape(1, d)

    gl, llcs = pl.pallas_call(
        _prep_body,
        out_shape=[
            jax.ShapeDtypeStruct((d, 2 * c), jnp.bfloat16),
            jax.ShapeDtypeStruct((1, d), jnp.float32),
        ],
    )(param_table, lg2)

    lpmf, pq, bit = pl.pallas_call(
        _vq_body,
        grid=(grid,),
        in_specs=[
            pl.BlockSpec((_BLK, c), lambda i: (i, 0)),
            pl.BlockSpec((_BLK, 1), lambda i: (i, 0)),
            pl.BlockSpec((1, d), lambda i: (0, 0)),
            pl.BlockSpec(memory_space=pl.ANY),
            pl.BlockSpec(memory_space=pl.ANY),
            pl.BlockSpec(memory_space=pl.ANY),
        ],
        out_specs=[
            pl.BlockSpec((_BLK, c), lambda i: (i, 0)),
            pl.BlockSpec((_BLK, c), lambda i: (i, 0)),
            pl.BlockSpec(memory_space=pltpu.SMEM),
        ],
        out_shape=[
            jax.ShapeDtypeStruct((tokens, c), jnp.float32),
            jax.ShapeDtypeStruct((tokens, c), jnp.float32),
            jax.ShapeDtypeStruct((1, 1), jnp.float32),
        ],
        scratch_shapes=[
            pltpu.VMEM((d, c), jnp.float32),
            pltpu.VMEM((d, 2 * c), jnp.bfloat16),
            pltpu.VMEM((1, d), jnp.float32),
            pltpu.VMEM((1, d), jnp.float32),
            pltpu.SemaphoreType.DMA,
        ],
        compiler_params=pltpu.CompilerParams(
            dimension_semantics=("arbitrary",),
        ),
    )(p2, pn, tn, param_table, gl, llcs)

    return (lpmf.reshape(a, b, c), pq.reshape(a, b, c), bit[0, 0])


# speedup vs baseline: 1.0561x; 1.0382x over previous
import math

import jax
import jax.numpy as jnp
from jax.experimental import pallas as pl
from jax.experimental.pallas import tpu as pltpu

_BLK = 512
_H = 256


def _prep_body(tab_ref, logit_ref, gl_ref, llcs_ref):
    c = tab_ref.shape[1]
    tab = tab_ref[...]
    mx = jnp.max(tab, axis=1, keepdims=True)
    ex = jnp.exp(tab - mx)
    ls = tab - (jnp.log(jnp.sum(ex, axis=1, keepdims=True)) + mx)
    lg = logit_ref[...]
    ml = jnp.max(lg)
    llc = lg - (jnp.log(jnp.sum(jnp.exp(lg - ml))) + ml)
    llcs_ref[...] = llc * (-1.0 / math.log(2.0))
    gl_ref[:, :c] = tab.astype(jnp.bfloat16)
    gl_ref[:, c:] = ls.astype(jnp.bfloat16)


def _vq_body(p_ref, pn_ref, tn_ref, tab_hbm, gl_hbm, llcs_hbm,
             lpmf_ref, pq_ref, bit_ref,
             tab_v, gl_v, llcs_v, cnt_v, sem):
    i = pl.program_id(0)
    nblk = pl.num_programs(0)
    d = tab_v.shape[0]
    c = tab_v.shape[1]

    @pl.when(i == 0)
    def _load_tables():
        c1 = pltpu.make_async_copy(tab_hbm, tab_v, sem)
        c2 = pltpu.make_async_copy(gl_hbm, gl_v, sem)
        c3 = pltpu.make_async_copy(llcs_hbm, llcs_v, sem)
        c1.start(); c2.start(); c3.start()
        c1.wait(); c2.wait(); c3.wait()
        cnt_v[...] = jnp.zeros((1, d), jnp.float32)

    tn = tn_ref[...]
    pncol = pn_ref[...].reshape(1, _BLK).T
    for h in range(_BLK // _H):
        p = jnp.clip(p_ref[pl.ds(h * _H, _H), :], -1.0, 1.0)
        pn = pncol[h * _H:(h + 1) * _H, :]
        scores = jax.lax.dot_general(
            p, tab_v[...], (((1,), (1,)), ((), ())),
            preferred_element_type=jnp.float32)
        dist = (tn + pn) - 2.0 * scores
        idx = jnp.argmin(dist, axis=1)
        iota = jax.lax.broadcasted_iota(jnp.int32, dist.shape, 1)
        ohf = (iota == idx[:, None]).astype(jnp.float32)
        cnt_v[...] += jnp.sum(ohf, axis=0, keepdims=True)
        qg = jax.lax.dot_general(
            ohf.astype(jnp.bfloat16), gl_v[...], (((1,), (0,)), ((), ())),
            preferred_element_type=jnp.float32)
        pq_ref[pl.ds(h * _H, _H), :] = qg[:, :c]
        lpmf_ref[pl.ds(h * _H, _H), :] = qg[:, c:]

    @pl.when(i == nblk - 1)
    def _finish():
        bit_ref[0, 0] = jnp.sum(cnt_v[...] * llcs_v[...])


def kernel(params, param_table, logits):
    a, b, c = params.shape
    d = param_table.shape[0]
    tokens = a * b
    p2 = params.reshape(tokens, c)
    lg2 = logits.reshape(1, d)
    grid = tokens // _BLK

    pclip = jnp.clip(p2, -1.0, 1.0)
    pn = jnp.sum(pclip ** 2, axis=-1).reshape(grid, 1, _BLK)
    tn = jnp.sum(param_table ** 2, axis=-1).reshape(1, d)

    gl, llcs = pl.pallas_call(
        _prep_body,
        out_shape=[
            jax.ShapeDtypeStruct((d, 2 * c), jnp.bfloat16),
            jax.ShapeDtypeStruct((1, d), jnp.float32),
        ],
    )(param_table, lg2)

    lpmf, pq, bit = pl.pallas_call(
        _vq_body,
        grid=(grid,),
        in_specs=[
            pl.BlockSpec((_BLK, c), lambda i: (i, 0)),
            pl.BlockSpec((1, 1, _BLK), lambda i: (i, 0, 0)),
            pl.BlockSpec((1, d), lambda i: (0, 0)),
            pl.BlockSpec(memory_space=pl.ANY),
            pl.BlockSpec(memory_space=pl.ANY),
            pl.BlockSpec(memory_space=pl.ANY),
        ],
        out_specs=[
            pl.BlockSpec((_BLK, c), lambda i: (i, 0)),
            pl.BlockSpec((_BLK, c), lambda i: (i, 0)),
            pl.BlockSpec(memory_space=pltpu.SMEM),
        ],
        out_shape=[
            jax.ShapeDtypeStruct((tokens, c), jnp.float32),
            jax.ShapeDtypeStruct((tokens, c), jnp.float32),
            jax.ShapeDtypeStruct((1, 1), jnp.float32),
        ],
        scratch_shapes=[
            pltpu.VMEM((d, c), jnp.float32),
            pltpu.VMEM((d, 2 * c), jnp.bfloat16),
            pltpu.VMEM((1, d), jnp.float32),
            pltpu.VMEM((1, d), jnp.float32),
            pltpu.SemaphoreType.DMA,
        ],
        compiler_params=pltpu.CompilerParams(
            dimension_semantics=("arbitrary",),
        ),
    )(p2, pn, tn, param_table, gl, llcs)

    return (lpmf.reshape(a, b, c), pq.reshape(a, b, c), bit[0, 0])


# speedup vs baseline: 1.1247x; 1.0649x over previous
import math

import jax
import jax.numpy as jnp
from jax.experimental import pallas as pl
from jax.experimental.pallas import tpu as pltpu

_BLK = 1024
_H = 256


def _prep_body(tab_ref, logit_ref, tab2_ref, gl_ref, llcs_ref):
    c = tab_ref.shape[1]
    tab = tab_ref[...]
    tab2_ref[...] = tab + tab
    mx = jnp.max(tab, axis=1, keepdims=True)
    ex = jnp.exp(tab - mx)
    ls = tab - (jnp.log(jnp.sum(ex, axis=1, keepdims=True)) + mx)
    lg = logit_ref[...]
    ml = jnp.max(lg)
    llc = lg - (jnp.log(jnp.sum(jnp.exp(lg - ml))) + ml)
    llcs_ref[...] = llc * (-1.0 / math.log(2.0))
    gl_ref[:, :c] = tab.astype(jnp.bfloat16)
    gl_ref[:, c:] = ls.astype(jnp.bfloat16)


def _vq_body(p_ref, pn_ref, tn_ref, tab_hbm, gl_hbm, llcs_hbm,
             lpmf_ref, pq_ref, bit_ref,
             tab_v, gl_v, llcs_v, cnt_v, sem):
    i = pl.program_id(0)
    nblk = pl.num_programs(0)
    d = tab_v.shape[0]
    c = tab_v.shape[1]

    @pl.when(i == 0)
    def _load_tables():
        c1 = pltpu.make_async_copy(tab_hbm, tab_v, sem)
        c2 = pltpu.make_async_copy(gl_hbm, gl_v, sem)
        c3 = pltpu.make_async_copy(llcs_hbm, llcs_v, sem)
        c1.start(); c2.start(); c3.start()
        c1.wait(); c2.wait(); c3.wait()
        cnt_v[...] = jnp.zeros((1, d), jnp.float32)

    tn = tn_ref[...]
    pncol = pn_ref[...].reshape(1, _BLK).T
    for h in range(_BLK // _H):
        p = jnp.clip(p_ref[pl.ds(h * _H, _H), :], -1.0, 1.0)
        pn = pncol[h * _H:(h + 1) * _H, :]
        s2 = jax.lax.dot_general(
            p, tab_v[...], (((1,), (1,)), ((), ())),
            preferred_element_type=jnp.float32)
        dist = (tn + pn) - s2
        idx = jnp.argmin(dist, axis=1)
        iota = jax.lax.broadcasted_iota(jnp.int32, dist.shape, 1)
        ohf = (iota == idx[:, None]).astype(jnp.float32)
        cnt_v[...] += jnp.sum(ohf, axis=0, keepdims=True)
        qg = jax.lax.dot_general(
            ohf.astype(jnp.bfloat16), gl_v[...], (((1,), (0,)), ((), ())),
            preferred_element_type=jnp.float32)
        pq_ref[pl.ds(h * _H, _H), :] = qg[:, :c]
        lpmf_ref[pl.ds(h * _H, _H), :] = qg[:, c:]

    @pl.when(i == nblk - 1)
    def _finish():
        bit_ref[0, 0] = jnp.sum(cnt_v[...] * llcs_v[...])


def kernel(params, param_table, logits):
    a, b, c = params.shape
    d = param_table.shape[0]
    tokens = a * b
    p2 = params.reshape(tokens, c)
    lg2 = logits.reshape(1, d)
    grid = tokens // _BLK

    pclip = jnp.clip(p2, -1.0, 1.0)
    pn = jnp.sum(pclip ** 2, axis=-1).reshape(grid, 1, _BLK)
    tn = jnp.sum(param_table ** 2, axis=-1).reshape(1, d)

    tab2, gl, llcs = pl.pallas_call(
        _prep_body,
        out_shape=[
            jax.ShapeDtypeStruct((d, c), jnp.float32),
            jax.ShapeDtypeStruct((d, 2 * c), jnp.bfloat16),
            jax.ShapeDtypeStruct((1, d), jnp.float32),
        ],
    )(param_table, lg2)

    lpmf, pq, bit = pl.pallas_call(
        _vq_body,
        grid=(grid,),
        in_specs=[
            pl.BlockSpec((_BLK, c), lambda i: (i, 0)),
            pl.BlockSpec((1, 1, _BLK), lambda i: (i, 0, 0)),
            pl.BlockSpec((1, d), lambda i: (0, 0)),
            pl.BlockSpec(memory_space=pl.ANY),
            pl.BlockSpec(memory_space=pl.ANY),
            pl.BlockSpec(memory_space=pl.ANY),
        ],
        out_specs=[
            pl.BlockSpec((_BLK, c), lambda i: (i, 0)),
            pl.BlockSpec((_BLK, c), lambda i: (i, 0)),
            pl.BlockSpec(memory_space=pltpu.SMEM),
        ],
        out_shape=[
            jax.ShapeDtypeStruct((tokens, c), jnp.float32),
            jax.ShapeDtypeStruct((tokens, c), jnp.float32),
            jax.ShapeDtypeStruct((1, 1), jnp.float32),
        ],
        scratch_shapes=[
            pltpu.VMEM((d, c), jnp.float32),
            pltpu.VMEM((d, 2 * c), jnp.bfloat16),
            pltpu.VMEM((1, d), jnp.float32),
            pltpu.VMEM((1, d), jnp.float32),
            pltpu.SemaphoreType.DMA,
        ],
        compiler_params=pltpu.CompilerParams(
            dimension_semantics=("arbitrary",),
        ),
    )(p2, pn, tn, tab2, gl, llcs)

    return (lpmf.reshape(a, b, c), pq.reshape(a, b, c), bit[0, 0])
